# baseline (speedup 1.0000x reference)
import functools

import jax
import jax.numpy as jnp
from jax import lax
from jax.experimental import pallas as pl
from jax.experimental.pallas import tpu as pltpu
from jax.experimental.pallas import tpu_sc as plsc

NUM_CLASSES = 10
NC = 2
NS = 16
L = 16
NW = NC * NS
CHUNK = 16384


def _mesh():
    return plsc.VectorSubcoreMesh(core_axis_name="c", subcore_axis_name="s")


def _wid():
    return lax.axis_index("s") * NC + lax.axis_index("c")


def _minmax_pass(n_per):
    n_chunks = n_per // CHUNK

    @functools.partial(
        pl.kernel,
        out_type=jax.ShapeDtypeStruct((NW, 2, L), jnp.float32),
        mesh=_mesh(),
        compiler_params=pltpu.CompilerParams(needs_layout_passes=False),
        scratch_types=[
            pltpu.VMEM((CHUNK,), jnp.float32),
            pltpu.VMEM((CHUNK,), jnp.float32),
            pltpu.VMEM((2, L), jnp.float32),
            pltpu.SemaphoreType.DMA,
            pltpu.SemaphoreType.DMA,
        ],
    )
    def body(x_hbm, mm_hbm, in0, in1, mm_v, sem0, sem1):
        wid = _wid()
        base = wid * n_per
        bufs = (in0, in1)
        sems = (sem0, sem1)

        copies = {}
        copies[0] = pltpu.async_copy(
            x_hbm.at[pl.ds(base, CHUNK)], bufs[0], sems[0])

        mn = jnp.full((L,), jnp.inf, jnp.float32)
        mx = jnp.full((L,), -jnp.inf, jnp.float32)

        for c in range(n_chunks):
            copies.pop(c).wait()
            if c + 1 < n_chunks:
                copies[c + 1] = pltpu.async_copy(
                    x_hbm.at[pl.ds(base + (c + 1) * CHUNK, CHUNK)],
                    bufs[(c + 1) % 2], sems[(c + 1) % 2])
            buf = bufs[c % 2]

            def red(i, carry, buf=buf):
                m0, m1 = carry
                v = buf[pl.ds(i * L, L)]
                return jnp.minimum(m0, v), jnp.maximum(m1, v)

            mn, mx = lax.fori_loop(0, CHUNK // L, red, (mn, mx))

        mm_v[0, :] = mn
        mm_v[1, :] = mx
        pltpu.sync_copy(mm_v, mm_hbm.at[wid])

    return body


def _assign_pass(n_per):
    n_chunks = n_per // CHUNK

    @functools.partial(
        pl.kernel,
        out_type=jax.ShapeDtypeStruct((n_per * NW,), jnp.int32),
        mesh=_mesh(),
        compiler_params=pltpu.CompilerParams(needs_layout_passes=False),
        scratch_types=[
            pltpu.VMEM((NW, 2, L), jnp.float32),
            pltpu.VMEM((L,), jnp.float32),
            pltpu.VMEM((L,), jnp.float32),
            pltpu.VMEM((CHUNK,), jnp.float32),
            pltpu.VMEM((CHUNK,), jnp.float32),
            pltpu.VMEM((CHUNK,), jnp.int32),
            pltpu.VMEM((CHUNK,), jnp.int32),
            pltpu.SemaphoreType.DMA,
            pltpu.SemaphoreType.DMA,
            pltpu.SemaphoreType.DMA,
            pltpu.SemaphoreType.DMA,
        ],
    )
    def body(x_hbm, mm_hbm, out_hbm, mm_v, mn_v, mx_v, in0, in1, o0, o1,
             si0, si1, so0, so1):
        wid = _wid()
        base = wid * n_per
        ibufs = (in0, in1)
        isems = (si0, si1)
        obufs = (o0, o1)
        osems = (so0, so1)

        pltpu.sync_copy(mm_hbm, mm_v)
        mn = mm_v[0, 0, :]
        mx = mm_v[0, 1, :]
        for t in range(1, NW):
            mn = jnp.minimum(mn, mm_v[t, 0, :])
            mx = jnp.maximum(mx, mm_v[t, 1, :])
        iota = lax.iota(jnp.int32, L)
        for shift in (8, 4, 2, 1):
            mn_v[:] = mn
            mx_v[:] = mx
            mn = jnp.minimum(mn, plsc.load_gather(mn_v, [iota ^ shift]))
            mx = jnp.maximum(mx, plsc.load_gather(mx_v, [iota ^ shift]))
        lo = mn + jnp.float32(1e-6)
        hi = mx - jnp.float32(1e-6)
        inv_step = jnp.float32(NUM_CLASSES - 2) / (hi - lo)

        in_copies = {}
        out_copies = {}
        in_copies[0] = pltpu.async_copy(
            x_hbm.at[pl.ds(base, CHUNK)], ibufs[0], isems[0])

        for c in range(n_chunks):
            in_copies.pop(c).wait()
            if c + 1 < n_chunks:
                in_copies[c + 1] = pltpu.async_copy(
                    x_hbm.at[pl.ds(base + (c + 1) * CHUNK, CHUNK)],
                    ibufs[(c + 1) % 2], isems[(c + 1) % 2])
            if c >= 2:
                out_copies.pop(c - 2).wait()
            ibuf = ibufs[c % 2]
            obuf = obufs[c % 2]

            def compute(i, _, ibuf=ibuf, obuf=obuf):
                v = ibuf[pl.ds(i * L, L)]
                q = (v - lo) * inv_step
                qi = q.astype(jnp.int32)
                qf = qi.astype(jnp.float32)
                lab = jnp.where(q > qf, qi + 1, qi)
                lab = jnp.minimum(jnp.maximum(lab, 0), NUM_CLASSES - 1)
                obuf[pl.ds(i * L, L)] = lab
                return 0

            lax.fori_loop(0, CHUNK // L, compute, 0)

            out_copies[c] = pltpu.async_copy(
                obuf, out_hbm.at[pl.ds(base + c * CHUNK, CHUNK)],
                osems[c % 2])

        for c in (n_chunks - 2, n_chunks - 1):
            if c >= 0:
                out_copies.pop(c).wait()

    return body


def kernel(input):
    n = input.shape[0]
    n_per = n // NW
    mm = _minmax_pass(n_per)(input)
    return _assign_pass(n_per)(input, mm)


# speedup vs baseline: 2.1846x; 2.1846x over previous
import functools

import jax
import jax.numpy as jnp
from jax import lax
from jax.experimental import pallas as pl
from jax.experimental.pallas import tpu as pltpu
from jax.experimental.pallas import tpu_sc as plsc

NUM_CLASSES = 10
NC = 2
NS = 16
L = 16
NW = NC * NS
CHUNK = 16384


def _mesh():
    return plsc.VectorSubcoreMesh(core_axis_name="c", subcore_axis_name="s")


def _wid():
    return lax.axis_index("s") * NC + lax.axis_index("c")


def _minmax_pass(n_per):
    n_chunks = n_per // CHUNK

    @functools.partial(
        pl.kernel,
        out_type=jax.ShapeDtypeStruct((NW, 2, L), jnp.float32),
        mesh=_mesh(),
        compiler_params=pltpu.CompilerParams(needs_layout_passes=False),
        scratch_types=[
            pltpu.VMEM((CHUNK,), jnp.float32),
            pltpu.VMEM((CHUNK,), jnp.float32),
            pltpu.VMEM((2, L), jnp.float32),
            pltpu.SemaphoreType.DMA,
            pltpu.SemaphoreType.DMA,
        ],
    )
    def body(x_hbm, mm_hbm, in0, in1, mm_v, sem0, sem1):
        wid = _wid()
        base = wid * n_per
        bufs = (in0, in1)
        sems = (sem0, sem1)

        copies = {}
        copies[0] = pltpu.async_copy(
            x_hbm.at[pl.ds(base, CHUNK)], bufs[0], sems[0])

        K = 8
        carry = (tuple(jnp.full((L,), jnp.inf, jnp.float32)
                       for _ in range(K))
                 + tuple(jnp.full((L,), -jnp.inf, jnp.float32)
                         for _ in range(K)))

        for c in range(n_chunks):
            copies.pop(c).wait()
            if c + 1 < n_chunks:
                copies[c + 1] = pltpu.async_copy(
                    x_hbm.at[pl.ds(base + (c + 1) * CHUNK, CHUNK)],
                    bufs[(c + 1) % 2], sems[(c + 1) % 2])
            buf = bufs[c % 2]

            def red(i, carry, buf=buf):
                vs = [buf[pl.ds(i * (K * L) + k * L, L)] for k in range(K)]
                return (tuple(jnp.minimum(carry[k], vs[k])
                              for k in range(K))
                        + tuple(jnp.maximum(carry[K + k], vs[k])
                                for k in range(K)))

            carry = lax.fori_loop(0, CHUNK // (K * L), red, carry)

        mn = carry[0]
        mx = carry[K]
        for k in range(1, K):
            mn = jnp.minimum(mn, carry[k])
            mx = jnp.maximum(mx, carry[K + k])
        mm_v[0, :] = mn
        mm_v[1, :] = mx
        pltpu.sync_copy(mm_v, mm_hbm.at[wid])

    return body


def _assign_pass(n_per):
    n_chunks = n_per // CHUNK

    @functools.partial(
        pl.kernel,
        out_type=jax.ShapeDtypeStruct((n_per * NW,), jnp.int32),
        mesh=_mesh(),
        compiler_params=pltpu.CompilerParams(needs_layout_passes=False),
        scratch_types=[
            pltpu.VMEM((NW, 2, L), jnp.float32),
            pltpu.VMEM((L,), jnp.float32),
            pltpu.VMEM((L,), jnp.float32),
            pltpu.VMEM((CHUNK,), jnp.float32),
            pltpu.VMEM((CHUNK,), jnp.float32),
            pltpu.VMEM((CHUNK,), jnp.int32),
            pltpu.VMEM((CHUNK,), jnp.int32),
            pltpu.SemaphoreType.DMA,
            pltpu.SemaphoreType.DMA,
            pltpu.SemaphoreType.DMA,
            pltpu.SemaphoreType.DMA,
        ],
    )
    def body(x_hbm, mm_hbm, out_hbm, mm_v, mn_v, mx_v, in0, in1, o0, o1,
             si0, si1, so0, so1):
        wid = _wid()
        base = wid * n_per
        ibufs = (in0, in1)
        isems = (si0, si1)
        obufs = (o0, o1)
        osems = (so0, so1)

        pltpu.sync_copy(mm_hbm, mm_v)
        mn = mm_v[0, 0, :]
        mx = mm_v[0, 1, :]
        for t in range(1, NW):
            mn = jnp.minimum(mn, mm_v[t, 0, :])
            mx = jnp.maximum(mx, mm_v[t, 1, :])
        iota = lax.iota(jnp.int32, L)
        for shift in (8, 4, 2, 1):
            mn_v[:] = mn
            mx_v[:] = mx
            mn = jnp.minimum(mn, plsc.load_gather(mn_v, [iota ^ shift]))
            mx = jnp.maximum(mx, plsc.load_gather(mx_v, [iota ^ shift]))
        lo = mn + jnp.float32(1e-6)
        hi = mx - jnp.float32(1e-6)
        inv_step = jnp.float32(NUM_CLASSES - 2) / (hi - lo)
        bias = jnp.float32(0.99999988) - lo * inv_step

        in_copies = {}
        out_copies = {}
        in_copies[0] = pltpu.async_copy(
            x_hbm.at[pl.ds(base, CHUNK)], ibufs[0], isems[0])

        for c in range(n_chunks):
            in_copies.pop(c).wait()
            if c + 1 < n_chunks:
                in_copies[c + 1] = pltpu.async_copy(
                    x_hbm.at[pl.ds(base + (c + 1) * CHUNK, CHUNK)],
                    ibufs[(c + 1) % 2], isems[(c + 1) % 2])
            if c >= 2:
                out_copies.pop(c - 2).wait()
            ibuf = ibufs[c % 2]
            obuf = obufs[c % 2]

            @plsc.parallel_loop(0, CHUNK, step=L, unroll=8)
            def compute(i, ibuf=ibuf, obuf=obuf):
                q = ibuf[pl.ds(i, L)] * inv_step + bias
                q = jnp.maximum(q, jnp.float32(0.0))
                q = jnp.minimum(q, jnp.float32(NUM_CLASSES - 0.5))
                obuf[pl.ds(i, L)] = q.astype(jnp.int32)

            out_copies[c] = pltpu.async_copy(
                obuf, out_hbm.at[pl.ds(base + c * CHUNK, CHUNK)],
                osems[c % 2])

        for c in (n_chunks - 2, n_chunks - 1):
            if c >= 0:
                out_copies.pop(c).wait()

    return body


def kernel(input):
    n = input.shape[0]
    n_per = n // NW
    mm = _minmax_pass(n_per)(input)
    return _assign_pass(n_per)(input, mm)


# speedup vs baseline: 2.4910x; 1.1402x over previous
import functools

import jax
import jax.numpy as jnp
from jax import lax
from jax.experimental import pallas as pl
from jax.experimental.pallas import tpu as pltpu
from jax.experimental.pallas import tpu_sc as plsc

NUM_CLASSES = 10
NC = 2
NS = 16
L = 16
NW = NC * NS
CHUNK = 16384


def _mesh():
    return plsc.VectorSubcoreMesh(core_axis_name="c", subcore_axis_name="s")


def _wid():
    return lax.axis_index("s") * NC + lax.axis_index("c")


def _minmax_pass(n_per):
    chunk = 2 * CHUNK
    n_chunks = n_per // chunk

    @functools.partial(
        pl.kernel,
        out_type=jax.ShapeDtypeStruct((NW, 2, L), jnp.float32),
        mesh=_mesh(),
        compiler_params=pltpu.CompilerParams(needs_layout_passes=False),
        scratch_types=[
            pltpu.VMEM((chunk,), jnp.float32),
            pltpu.VMEM((chunk,), jnp.float32),
            pltpu.VMEM((2, L), jnp.float32),
            pltpu.SemaphoreType.DMA,
            pltpu.SemaphoreType.DMA,
        ],
    )
    def body(x_hbm, mm_hbm, in0, in1, mm_v, sem0, sem1):
        wid = _wid()
        base = wid * n_per
        bufs = (in0, in1)
        sems = (sem0, sem1)

        copies = {}
        copies[0] = pltpu.async_copy(
            x_hbm.at[pl.ds(base, chunk)], bufs[0], sems[0])

        K = 16
        carry = (tuple(jnp.full((L,), jnp.inf, jnp.float32)
                       for _ in range(K))
                 + tuple(jnp.full((L,), -jnp.inf, jnp.float32)
                         for _ in range(K)))

        for c in range(n_chunks):
            copies.pop(c).wait()
            if c + 1 < n_chunks:
                copies[c + 1] = pltpu.async_copy(
                    x_hbm.at[pl.ds(base + (c + 1) * chunk, chunk)],
                    bufs[(c + 1) % 2], sems[(c + 1) % 2])
            buf = bufs[c % 2]

            def red(i, carry, buf=buf):
                vs = [buf[pl.ds(i * (K * L) + k * L, L)] for k in range(K)]
                return (tuple(jnp.minimum(carry[k], vs[k])
                              for k in range(K))
                        + tuple(jnp.maximum(carry[K + k], vs[k])
                                for k in range(K)))

            carry = lax.fori_loop(0, chunk // (K * L), red, carry)

        mn = carry[0]
        mx = carry[K]
        for k in range(1, K):
            mn = jnp.minimum(mn, carry[k])
            mx = jnp.maximum(mx, carry[K + k])
        mm_v[0, :] = mn
        mm_v[1, :] = mx
        pltpu.sync_copy(mm_v, mm_hbm.at[wid])

    return body


def _assign_pass(n_per):
    n_chunks = n_per // CHUNK

    @functools.partial(
        pl.kernel,
        out_type=jax.ShapeDtypeStruct((n_per * NW,), jnp.int32),
        mesh=_mesh(),
        compiler_params=pltpu.CompilerParams(needs_layout_passes=False),
        scratch_types=[
            pltpu.VMEM((NW, 2, L), jnp.float32),
            pltpu.VMEM((L,), jnp.float32),
            pltpu.VMEM((L,), jnp.float32),
            pltpu.VMEM((CHUNK,), jnp.float32),
            pltpu.VMEM((CHUNK,), jnp.float32),
            pltpu.VMEM((CHUNK,), jnp.float32),
            pltpu.VMEM((CHUNK,), jnp.int32),
            pltpu.VMEM((CHUNK,), jnp.int32),
            pltpu.SemaphoreType.DMA,
            pltpu.SemaphoreType.DMA,
            pltpu.SemaphoreType.DMA,
            pltpu.SemaphoreType.DMA,
            pltpu.SemaphoreType.DMA,
        ],
    )
    def body(x_hbm, mm_hbm, out_hbm, mm_v, mn_v, mx_v, in0, in1, in2,
             o0, o1, si0, si1, si2, so0, so1):
        wid = _wid()
        base = wid * n_per
        ibufs = (in0, in1, in2)
        isems = (si0, si1, si2)
        obufs = (o0, o1)
        osems = (so0, so1)

        pltpu.sync_copy(mm_hbm, mm_v)
        mn = mm_v[0, 0, :]
        mx = mm_v[0, 1, :]
        for t in range(1, NW):
            mn = jnp.minimum(mn, mm_v[t, 0, :])
            mx = jnp.maximum(mx, mm_v[t, 1, :])
        iota = lax.iota(jnp.int32, L)
        for shift in (8, 4, 2, 1):
            mn_v[:] = mn
            mx_v[:] = mx
            mn = jnp.minimum(mn, plsc.load_gather(mn_v, [iota ^ shift]))
            mx = jnp.maximum(mx, plsc.load_gather(mx_v, [iota ^ shift]))
        lo = mn + jnp.float32(1e-6)
        hi = mx - jnp.float32(1e-6)
        inv_step = jnp.float32(NUM_CLASSES - 2) / (hi - lo)
        bias = jnp.float32(0.99999988) - lo * inv_step

        in_copies = {}
        out_copies = {}
        for p in range(2):
            in_copies[p] = pltpu.async_copy(
                x_hbm.at[pl.ds(base + p * CHUNK, CHUNK)],
                ibufs[p], isems[p])

        for c in range(n_chunks):
            in_copies.pop(c).wait()
            if c + 2 < n_chunks:
                in_copies[c + 2] = pltpu.async_copy(
                    x_hbm.at[pl.ds(base + (c + 2) * CHUNK, CHUNK)],
                    ibufs[(c + 2) % 3], isems[(c + 2) % 3])
            if c >= 2:
                out_copies.pop(c - 2).wait()
            ibuf = ibufs[c % 3]
            obuf = obufs[c % 2]

            @plsc.parallel_loop(0, CHUNK, step=L, unroll=16)
            def compute(i, ibuf=ibuf, obuf=obuf):
                q = ibuf[pl.ds(i, L)] * inv_step + bias
                q = jnp.maximum(q, jnp.float32(0.0))
                q = jnp.minimum(q, jnp.float32(NUM_CLASSES - 0.5))
                obuf[pl.ds(i, L)] = q.astype(jnp.int32)

            out_copies[c] = pltpu.async_copy(
                obuf, out_hbm.at[pl.ds(base + c * CHUNK, CHUNK)],
                osems[c % 2])

        for c in (n_chunks - 2, n_chunks - 1):
            if c >= 0:
                out_copies.pop(c).wait()

    return body


def kernel(input):
    n = input.shape[0]
    n_per = n // NW
    mm = _minmax_pass(n_per)(input)
    return _assign_pass(n_per)(input, mm)


# speedup vs baseline: 2.7835x; 1.1174x over previous
import functools

import jax
import jax.numpy as jnp
from jax import lax
from jax.experimental import pallas as pl
from jax.experimental.pallas import tpu as pltpu
from jax.experimental.pallas import tpu_sc as plsc

NUM_CLASSES = 10
NC = 2
NS = 16
L = 16
NW = NC * NS
CHUNK = 16384


def _mesh():
    return plsc.VectorSubcoreMesh(core_axis_name="c", subcore_axis_name="s")


def _wid():
    return lax.axis_index("s") * NC + lax.axis_index("c")


def _minmax_pass_tc(n):
    GRID = 32
    rows = n // 128 // GRID

    def body(x_ref, out_ref, acc_ref):
        i = pl.program_id(0)

        @pl.when(i == 0)
        def _():
            acc_ref[0, :] = jnp.full((128,), jnp.inf, jnp.float32)
            acc_ref[1, :] = jnp.full((128,), -jnp.inf, jnp.float32)

        xb = x_ref[...]
        acc_ref[0, :] = jnp.minimum(acc_ref[0, :], jnp.min(xb, axis=(0, 1)))
        acc_ref[1, :] = jnp.maximum(acc_ref[1, :], jnp.max(xb, axis=(0, 1)))

        @pl.when(i == GRID - 1)
        def _():
            out_ref[...] = acc_ref[...]

    def run(x):
        xr = x.reshape(GRID, rows, 128)
        return pl.pallas_call(
            body,
            grid=(GRID,),
            in_specs=[pl.BlockSpec((1, rows, 128), lambda i: (i, 0, 0))],
            out_specs=pl.BlockSpec((2, 128), lambda i: (0, 0)),
            out_shape=jax.ShapeDtypeStruct((2, 128), jnp.float32),
            scratch_shapes=[pltpu.VMEM((2, 128), jnp.float32)],
        )(xr)

    return run


def _assign_pass(n_per):
    n_chunks = n_per // CHUNK

    @functools.partial(
        pl.kernel,
        out_type=jax.ShapeDtypeStruct((n_per * NW,), jnp.int32),
        mesh=_mesh(),
        compiler_params=pltpu.CompilerParams(needs_layout_passes=False),
        scratch_types=[
            pltpu.VMEM((2, 128), jnp.float32),
            pltpu.VMEM((L,), jnp.float32),
            pltpu.VMEM((L,), jnp.float32),
            pltpu.VMEM((CHUNK,), jnp.float32),
            pltpu.VMEM((CHUNK,), jnp.float32),
            pltpu.VMEM((CHUNK,), jnp.float32),
            pltpu.VMEM((CHUNK,), jnp.int32),
            pltpu.VMEM((CHUNK,), jnp.int32),
            pltpu.SemaphoreType.DMA,
            pltpu.SemaphoreType.DMA,
            pltpu.SemaphoreType.DMA,
            pltpu.SemaphoreType.DMA,
            pltpu.SemaphoreType.DMA,
        ],
    )
    def body(x_hbm, mm_hbm, out_hbm, mm_v, mn_v, mx_v, in0, in1, in2,
             o0, o1, si0, si1, si2, so0, so1):
        wid = _wid()
        base = wid * n_per
        ibufs = (in0, in1, in2)
        isems = (si0, si1, si2)
        obufs = (o0, o1)
        osems = (so0, so1)

        pltpu.sync_copy(mm_hbm, mm_v)
        mn = mm_v[0, pl.ds(0, L)]
        mx = mm_v[1, pl.ds(0, L)]
        for t in range(1, 128 // L):
            mn = jnp.minimum(mn, mm_v[0, pl.ds(t * L, L)])
            mx = jnp.maximum(mx, mm_v[1, pl.ds(t * L, L)])
        iota = lax.iota(jnp.int32, L)
        for shift in (8, 4, 2, 1):
            mn_v[:] = mn
            mx_v[:] = mx
            mn = jnp.minimum(mn, plsc.load_gather(mn_v, [iota ^ shift]))
            mx = jnp.maximum(mx, plsc.load_gather(mx_v, [iota ^ shift]))
        lo = mn + jnp.float32(1e-6)
        hi = mx - jnp.float32(1e-6)
        inv_step = jnp.float32(NUM_CLASSES - 2) / (hi - lo)
        bias = jnp.float32(0.99999988) - lo * inv_step

        in_copies = {}
        out_copies = {}
        for p in range(2):
            in_copies[p] = pltpu.async_copy(
                x_hbm.at[pl.ds(base + p * CHUNK, CHUNK)],
                ibufs[p], isems[p])

        for c in range(n_chunks):
            in_copies.pop(c).wait()
            if c + 2 < n_chunks:
                in_copies[c + 2] = pltpu.async_copy(
                    x_hbm.at[pl.ds(base + (c + 2) * CHUNK, CHUNK)],
                    ibufs[(c + 2) % 3], isems[(c + 2) % 3])
            if c >= 2:
                out_copies.pop(c - 2).wait()
            ibuf = ibufs[c % 3]
            obuf = obufs[c % 2]

            @plsc.parallel_loop(0, CHUNK, step=L, unroll=16)
            def compute(i, ibuf=ibuf, obuf=obuf):
                q = ibuf[pl.ds(i, L)] * inv_step + bias
                q = jnp.minimum(q, jnp.float32(NUM_CLASSES - 0.5))
                obuf[pl.ds(i, L)] = q.astype(jnp.int32)

            out_copies[c] = pltpu.async_copy(
                obuf, out_hbm.at[pl.ds(base + c * CHUNK, CHUNK)],
                osems[c % 2])

        for c in (n_chunks - 2, n_chunks - 1):
            if c >= 0:
                out_copies.pop(c).wait()

    return body


def kernel(input):
    n = input.shape[0]
    n_per = n // NW
    mm = _minmax_pass_tc(n)(input)
    return _assign_pass(n_per)(input, mm)
